# Initial kernel scaffold; baseline (speedup 1.0000x reference)
import jax
import jax.numpy as jnp
from jax.experimental import pallas as pl


def kernel(fg, emb_weight):
    raise NotImplementedError("write your pallas kernel here")


# speedup vs baseline: 1.0939x; 1.0939x over previous
import functools

import jax
import jax.numpy as jnp
from jax import lax
from jax.experimental import pallas as pl
from jax.experimental.pallas import tpu as pltpu
from jax.experimental.pallas import tpu_sc as plsc

_D = 32
_GATHER = 128
_CHUNK = 1024


def kernel(fg, emb_weight):
    batch, hist = fg.shape
    total = batch * hist
    idx_flat = fg.reshape(total).astype(jnp.int32)

    num_workers = 32
    per_worker = total // num_workers
    chunks = per_worker // _CHUNK
    gathers = _CHUNK // _GATHER

    mesh = plsc.VectorSubcoreMesh(core_axis_name="c", subcore_axis_name="s")

    @functools.partial(
        pl.kernel,
        mesh=mesh,
        out_type=jax.ShapeDtypeStruct((total, _D), jnp.float32),
        scratch_types=[
            pltpu.VMEM((_CHUNK,), jnp.int32),
            pltpu.VMEM((_CHUNK, _D), jnp.float32),
            pltpu.SemaphoreType.DMA,
        ],
        compiler_params=pltpu.CompilerParams(use_tc_tiling_on_sc=False),
    )
    def _gather_kernel(table_hbm, idx_hbm, out_hbm, idx_v, rows_v, sem):
        wid = lax.axis_index("s") * 2 + lax.axis_index("c")
        base = wid * per_worker

        def body(i, carry):
            off = base + i * _CHUNK
            pltpu.sync_copy(idx_hbm.at[pl.ds(off, _CHUNK)], idx_v)
            copies = []
            for j in range(gathers):
                copies.append(
                    pltpu.async_copy(
                        table_hbm.at[idx_v.at[pl.ds(j * _GATHER, _GATHER)]],
                        rows_v.at[pl.ds(j * _GATHER, _GATHER)],
                        sem,
                    )
                )
            for c in copies:
                c.wait()
            pltpu.sync_copy(rows_v, out_hbm.at[pl.ds(off, _CHUNK)])
            return carry

        lax.fori_loop(0, chunks, body, 0)

    out = _gather_kernel(emb_weight, idx_flat)
    return out.reshape(batch, hist, _D)


# speedup vs baseline: 1.1097x; 1.0144x over previous
import functools

import jax
import jax.numpy as jnp
from jax import lax
from jax.experimental import pallas as pl
from jax.experimental.pallas import tpu as pltpu
from jax.experimental.pallas import tpu_sc as plsc

_D = 32
_GATHER = 128
_CHUNK = 1280


def kernel(fg, emb_weight):
    batch, hist = fg.shape
    total = batch * hist
    idx_flat = fg.reshape(total).astype(jnp.int32)

    num_workers = 32
    per_worker = total // num_workers
    chunks = per_worker // _CHUNK
    gathers = _CHUNK // _GATHER

    mesh = plsc.VectorSubcoreMesh(core_axis_name="c", subcore_axis_name="s")

    @functools.partial(
        pl.kernel,
        mesh=mesh,
        out_type=jax.ShapeDtypeStruct((total, _D), jnp.float32),
        scratch_types=[
            pltpu.VMEM((per_worker,), jnp.int32),
            pltpu.VMEM((2, _CHUNK, _D), jnp.float32),
            pltpu.SemaphoreType.DMA,
            pltpu.SemaphoreType.DMA,
            pltpu.SemaphoreType.DMA,
        ],
        compiler_params=pltpu.CompilerParams(use_tc_tiling_on_sc=False),
    )
    def _gather_kernel(table_hbm, idx_hbm, out_hbm, idx_v, rows_v, gsem,
                       osem0, osem1):
        wid = lax.axis_index("s") * 2 + lax.axis_index("c")
        base = wid * per_worker
        osems = (osem0, osem1)

        pltpu.sync_copy(idx_hbm.at[pl.ds(base, per_worker)], idx_v)

        def out_copy(b, off):
            return pltpu.make_async_copy(
                rows_v.at[b], out_hbm.at[pl.ds(off, _CHUNK)], osems[b])

        def do_chunk(b, g):
            local = g * _CHUNK
            copies = []
            for j in range(gathers):
                copies.append(
                    pltpu.async_copy(
                        table_hbm.at[idx_v.at[pl.ds(local + j * _GATHER,
                                                    _GATHER)]],
                        rows_v.at[b, pl.ds(j * _GATHER, _GATHER)],
                        gsem,
                    )
                )
            for c in copies:
                c.wait()
            out_copy(b, base + local).start()

        def body(t, carry):
            for b in (0, 1):
                g = 2 * t + b

                @pl.when(t > 0)
                def _():
                    out_copy(b, base + (g - 2) * _CHUNK).wait()

                do_chunk(b, g)
            return carry

        lax.fori_loop(0, chunks // 2, body, 0)
        out_copy(0, base + (chunks - 2) * _CHUNK).wait()
        out_copy(1, base + (chunks - 1) * _CHUNK).wait()

    out = _gather_kernel(emb_weight, idx_flat)
    return out.reshape(batch, hist, _D)


# speedup vs baseline: 1.7907x; 1.6136x over previous
import functools

import jax
import jax.numpy as jnp
from jax import lax
from jax.experimental import pallas as pl
from jax.experimental.pallas import tpu as pltpu
from jax.experimental.pallas import tpu_sc as plsc

_D = 32
_KROWS = 16


def kernel(fg, emb_weight):
    batch, hist = fg.shape
    fg = fg.astype(jnp.int32)

    num_workers = 32
    rows_per_w = batch // num_workers
    chunks = rows_per_w // _KROWS

    mesh = plsc.VectorSubcoreMesh(core_axis_name="c", subcore_axis_name="s")

    @functools.partial(
        pl.kernel,
        mesh=mesh,
        out_type=jax.ShapeDtypeStruct((batch, hist, _D), jnp.float32),
        scratch_types=[
            pltpu.VMEM((rows_per_w, hist), jnp.int32),
            pltpu.VMEM((2, _KROWS, hist, _D), jnp.float32),
            pltpu.SemaphoreType.DMA,
            pltpu.SemaphoreType.DMA,
            pltpu.SemaphoreType.DMA,
        ],
        compiler_params=pltpu.CompilerParams(use_tc_tiling_on_sc=False),
    )
    def _gather_kernel(table_hbm, idx_hbm, out_hbm, idx_v, rows_v, gsem,
                       osem0, osem1):
        wid = lax.axis_index("s") * 2 + lax.axis_index("c")
        base = wid * rows_per_w
        osems = (osem0, osem1)

        pltpu.sync_copy(idx_hbm.at[pl.ds(base, rows_per_w)], idx_v)

        def out_copy(b, row0):
            return pltpu.make_async_copy(
                rows_v.at[b], out_hbm.at[pl.ds(row0, _KROWS)], osems[b])

        def do_chunk(b, g):
            local = g * _KROWS
            copies = []
            for j in range(_KROWS):
                copies.append(
                    pltpu.async_copy(
                        table_hbm.at[idx_v.at[local + j]],
                        rows_v.at[b, j],
                        gsem,
                    )
                )
            for c in copies:
                c.wait()
            out_copy(b, base + local).start()

        def body(t, carry):
            for b in (0, 1):
                g = 2 * t + b

                @pl.when(t > 0)
                def _():
                    out_copy(b, base + (g - 2) * _KROWS).wait()

                do_chunk(b, g)
            return carry

        lax.fori_loop(0, chunks // 2, body, 0)
        out_copy(0, base + (chunks - 2) * _KROWS).wait()
        out_copy(1, base + (chunks - 1) * _KROWS).wait()

    return _gather_kernel(emb_weight, fg)


# speedup vs baseline: 1.7970x; 1.0035x over previous
import functools

import jax
import jax.numpy as jnp
from jax import lax
from jax.experimental import pallas as pl
from jax.experimental.pallas import tpu as pltpu
from jax.experimental.pallas import tpu_sc as plsc

_D = 32
_KROWS = 32


def kernel(fg, emb_weight):
    batch, hist = fg.shape
    fg = fg.astype(jnp.int32)

    num_workers = 32
    rows_per_w = batch // num_workers
    chunks = rows_per_w // _KROWS

    mesh = plsc.VectorSubcoreMesh(core_axis_name="c", subcore_axis_name="s")

    @functools.partial(
        pl.kernel,
        mesh=mesh,
        out_type=jax.ShapeDtypeStruct((batch, hist, _D), jnp.float32),
        scratch_types=[
            pltpu.VMEM((rows_per_w, hist), jnp.int32),
            pltpu.VMEM((2, _KROWS, hist, _D), jnp.float32),
            pltpu.SemaphoreType.DMA,
            pltpu.SemaphoreType.DMA,
            pltpu.SemaphoreType.DMA,
        ],
        compiler_params=pltpu.CompilerParams(use_tc_tiling_on_sc=False),
    )
    def _gather_kernel(table_hbm, idx_hbm, out_hbm, idx_v, rows_v, gsem,
                       osem0, osem1):
        wid = lax.axis_index("s") * 2 + lax.axis_index("c")
        base = wid * rows_per_w
        osems = (osem0, osem1)

        pltpu.sync_copy(idx_hbm.at[pl.ds(base, rows_per_w)], idx_v)

        def out_copy(b, row0):
            return pltpu.make_async_copy(
                rows_v.at[b], out_hbm.at[pl.ds(row0, _KROWS)], osems[b])

        def do_chunk(b, g):
            local = g * _KROWS
            copies = []
            for j in range(_KROWS):
                copies.append(
                    pltpu.async_copy(
                        table_hbm.at[idx_v.at[local + j]],
                        rows_v.at[b, j],
                        gsem,
                    )
                )
            for c in copies:
                c.wait()
            out_copy(b, base + local).start()

        def body(t, carry):
            for b in (0, 1):
                g = 2 * t + b

                @pl.when(t > 0)
                def _():
                    out_copy(b, base + (g - 2) * _KROWS).wait()

                do_chunk(b, g)
            return carry

        lax.fori_loop(0, chunks // 2, body, 0)
        out_copy(0, base + (chunks - 2) * _KROWS).wait()
        out_copy(1, base + (chunks - 1) * _KROWS).wait()

    return _gather_kernel(emb_weight, fg)
